# baseline (speedup 1.0000x reference)
import functools

import jax
import jax.numpy as jnp
from jax import lax
from jax.experimental import pallas as pl
from jax.experimental.pallas import tpu as pltpu

_EPS = 1e-5
_BF = jnp.bfloat16
_TH = {256: 4, 128: 8, 64: 16, 32: 32, 16: 16}


def _halo_rows(a, th):
    n, h, w, c = a.shape
    zrow = jnp.zeros((n, 1, w, c), a.dtype)
    top = jnp.concatenate([zrow, a[:, th - 1:h - 1:th]], axis=1)
    bot = jnp.concatenate([a[:, th::th], zrow], axis=1)
    return top, bot


def _conv_body(top_ref, body_ref, bot_ref, scale_ref, shift_ref, w_ref,
               *refs, th, hh, ww, cin, cout, per_tap, affine, mode,
               alpha):
    b = pl.program_id(1)
    slab = jnp.concatenate(
        [top_ref[...], body_ref[...], bot_ref[...]], axis=1)
    if affine:
        t = slab.astype(jnp.float32) * scale_ref[...] + shift_ref[...]
        t = jnp.maximum(t, 0.0)
        g = lax.broadcasted_iota(jnp.int32, (1, th + 2, 1, 1), 1) + b * th - 1
        slab = jnp.where((g >= 0) & (g < hh), t, 0.0).astype(_BF)
    slab = jnp.pad(slab, ((0, 0), (0, 0), (1, 1), (0, 0)))
    m = slab.shape[0] * th * ww
    taps = [(dy, dx) for dy in range(3) for dx in range(3)]
    if per_tap:
        y = jnp.zeros((m, cout), jnp.float32)
        for t_i, (dy, dx) in enumerate(taps):
            p = slab[:, dy:dy + th, dx:dx + ww, :].reshape(m, cin)
            y = y + jnp.dot(p, w_ref[t_i * cin:(t_i + 1) * cin, :],
                            preferred_element_type=jnp.float32)
    else:
        cols = [slab[:, dy:dy + th, dx:dx + ww, :].reshape(m, cin)
                for dy, dx in taps]
        y = jnp.dot(jnp.concatenate(cols, axis=-1), w_ref[...],
                    preferred_element_type=jnp.float32)
    nblk = slab.shape[0]
    if mode == 'final':
        y = jnp.where(y >= 0, y, alpha * y)
        y = 1.0 / (1.0 + jnp.exp(-y))
        refs[0][...] = y.reshape(nblk, th, ww, cout)
    elif mode == 'fused':
        gamma_ref, beta_ref, z_ref = refs
        mean = jnp.sum(y, axis=0, keepdims=True) * (1.0 / m)
        yc = y - mean
        var = jnp.sum(yc * yc, axis=0, keepdims=True) * (1.0 / m)
        sc = gamma_ref[...] * lax.rsqrt(var + _EPS)
        t = yc * sc + beta_ref[...]
        t = jnp.maximum(t, 0.0)
        z_ref[...] = t.reshape(nblk, th, ww, cout).astype(z_ref.dtype)
    else:
        y_ref, s_ref, ss_ref = refs
        y_ref[...] = y.reshape(nblk, th, ww, cout).astype(y_ref.dtype)
        s_ref[...] = jnp.sum(y, axis=0).reshape(1, 1, cout)
        ss_ref[...] = jnp.sum(y * y, axis=0).reshape(1, 1, cout)


def _conv3x3(src, w_bf, scale=None, shift=None, *, mode='stats', alpha=0.0,
             y_dtype=_BF, gamma=None, beta=None):
    n, h, ww, cin = src.shape
    k, cout = w_bf.shape
    fused = mode == 'fused'
    th = h if fused else _TH[h]
    nb = h // th
    per_tap = cin >= 128
    affine = scale is not None
    if not affine:
        scale = jnp.zeros((1, 1, 1, cin), jnp.float32)
        shift = jnp.zeros((1, 1, 1, cin), jnp.float32)
    top, bot = _halo_rows(src, th)

    bn = n if fused else 1
    grid = (1, 1) if fused else (n, nb)
    halo_spec = pl.BlockSpec((bn, 1, ww, cin), lambda i, j: (i, j, 0, 0))
    body_spec = pl.BlockSpec((bn, th, ww, cin), lambda i, j: (i, j, 0, 0))
    vec_spec = pl.BlockSpec((1, 1, 1, cin), lambda i, j: (0, 0, 0, 0))
    w_spec = pl.BlockSpec((k, cout), lambda i, j: (0, 0))
    y_spec = pl.BlockSpec((bn, th, ww, cout), lambda i, j: (i, j, 0, 0))
    st_spec = pl.BlockSpec((1, 1, cout), lambda i, j: (i * nb + j, 0, 0))

    inputs = [top, src, bot, scale, shift, w_bf]
    in_specs = [halo_spec, body_spec, halo_spec, vec_spec, vec_spec, w_spec]
    if mode == 'final':
        out_shape = jax.ShapeDtypeStruct((n, h, ww, cout), jnp.float32)
        out_specs = y_spec
    elif fused:
        g_spec = pl.BlockSpec((1, cout), lambda i, j: (0, 0))
        inputs += [gamma.reshape(1, cout).astype(jnp.float32),
                   beta.reshape(1, cout).astype(jnp.float32)]
        in_specs += [g_spec, g_spec]
        out_shape = jax.ShapeDtypeStruct((n, h, ww, cout), _BF)
        out_specs = y_spec
    else:
        out_shape = (jax.ShapeDtypeStruct((n, h, ww, cout), y_dtype),
                     jax.ShapeDtypeStruct((n * nb, 1, cout), jnp.float32),
                     jax.ShapeDtypeStruct((n * nb, 1, cout), jnp.float32))
        out_specs = (y_spec, st_spec, st_spec)

    body = functools.partial(
        _conv_body, th=th, hh=h, ww=ww, cin=cin, cout=cout,
        per_tap=per_tap, affine=affine, mode=mode, alpha=alpha)
    return pl.pallas_call(
        body,
        grid=grid,
        in_specs=in_specs,
        out_specs=out_specs,
        out_shape=out_shape,
        compiler_params=pltpu.CompilerParams(
            dimension_semantics=("parallel", "parallel"),
            vmem_limit_bytes=100 * 1024 * 1024),
    )(*inputs)


def _bn_fold(stats, gamma, beta):
    y, s, ss = stats
    m = y.shape[0] * y.shape[1] * y.shape[2]
    pair = jnp.stack([s[:, 0, :], ss[:, 0, :]])
    tot, _ = lax.scan(lambda a, p: (a + p, None),
                      jnp.zeros_like(pair[:, 0]), pair.transpose(1, 0, 2))
    mean = tot[0] / m
    var = jnp.maximum(tot[1] / m - mean * mean, 0.0)
    sc = gamma * lax.rsqrt(var + _EPS)
    sh = beta - mean * sc
    return sc.reshape(1, 1, 1, -1), sh.reshape(1, 1, 1, -1)


def _apply_body(y_ref, sc_ref, sh_ref, z_ref):
    t = y_ref[...].astype(jnp.float32) * sc_ref[...] + sh_ref[...]
    z_ref[...] = jnp.maximum(t, 0.0).astype(z_ref.dtype)


def _apply(y, sc, sh):
    n, h, ww, c = y.shape
    th = _TH[h]
    blk = pl.BlockSpec((1, th, ww, c), lambda i, j: (i, j, 0, 0))
    vec = pl.BlockSpec((1, 1, 1, c), lambda i, j: (0, 0, 0, 0))
    return pl.pallas_call(
        _apply_body,
        grid=(n, h // th),
        in_specs=[blk, vec, vec],
        out_specs=blk,
        out_shape=jax.ShapeDtypeStruct((n, h, ww, c), _BF),
        compiler_params=pltpu.CompilerParams(
            dimension_semantics=("parallel", "parallel")),
    )(y, sc, sh)


def _pool2(z):
    n, h, w, c = z.shape
    return (z.astype(jnp.float32)
            .reshape(n, h // 2, 2, w // 2, 2, c).mean(axis=(2, 4))
            .astype(_BF))


def _up2(z):
    return jnp.repeat(jnp.repeat(z, 2, axis=1), 2, axis=2)


def kernel(x, enc1_1_w, enc1_1_gamma, enc1_1_beta, enc1_2_w, enc1_2_gamma, enc1_2_beta, enc2_1_w, enc2_1_gamma, enc2_1_beta, enc2_2_w, enc2_2_gamma, enc2_2_beta, enc3_1_w, enc3_1_gamma, enc3_1_beta, enc3_2_w, enc3_2_gamma, enc3_2_beta, enc4_1_w, enc4_1_gamma, enc4_1_beta, enc4_2_w, enc4_2_gamma, enc4_2_beta, enc5_1_w, enc5_1_gamma, enc5_1_beta, dec5_1_w, dec5_1_gamma, dec5_1_beta, dec4_2_w, dec4_2_gamma, dec4_2_beta, dec4_1_w, dec4_1_gamma, dec4_1_beta, dec3_2_w, dec3_2_gamma, dec3_2_beta, dec3_1_w, dec3_1_gamma, dec3_1_beta, dec2_2_w, dec2_2_gamma, dec2_2_beta, dec2_1_w, dec2_1_gamma, dec2_1_beta, dec1_2_w, dec1_2_gamma, dec1_2_beta, dec1_1_w):
    bf = lambda a: a.astype(_BF)
    xin = bf(jnp.transpose(x, (0, 2, 3, 1)))

    st = _conv3x3(xin, bf(enc1_1_w))
    sc, sh = _bn_fold(st, enc1_1_gamma, enc1_1_beta)
    st = _conv3x3(st[0], bf(enc1_2_w), sc, sh)
    sc, sh = _bn_fold(st, enc1_2_gamma, enc1_2_beta)
    z_enc1 = _apply(st[0], sc, sh)

    st = _conv3x3(_pool2(z_enc1), bf(enc2_1_w))
    sc, sh = _bn_fold(st, enc2_1_gamma, enc2_1_beta)
    st = _conv3x3(st[0], bf(enc2_2_w), sc, sh)
    sc, sh = _bn_fold(st, enc2_2_gamma, enc2_2_beta)
    z_enc2 = _apply(st[0], sc, sh)

    st = _conv3x3(_pool2(z_enc2), bf(enc3_1_w))
    sc, sh = _bn_fold(st, enc3_1_gamma, enc3_1_beta)
    st = _conv3x3(st[0], bf(enc3_2_w), sc, sh)
    sc, sh = _bn_fold(st, enc3_2_gamma, enc3_2_beta)
    z_enc3 = _apply(st[0], sc, sh)

    st = _conv3x3(_pool2(z_enc3), bf(enc4_1_w))
    sc, sh = _bn_fold(st, enc4_1_gamma, enc4_1_beta)
    st = _conv3x3(st[0], bf(enc4_2_w), sc, sh)
    sc, sh = _bn_fold(st, enc4_2_gamma, enc4_2_beta)
    z_enc4 = _apply(st[0], sc, sh)

    z_enc5 = _conv3x3(_pool2(z_enc4), bf(enc5_1_w), mode='fused',
                      gamma=enc5_1_gamma, beta=enc5_1_beta)

    z_dec5 = _conv3x3(z_enc5, bf(dec5_1_w), mode='fused',
                      gamma=dec5_1_gamma, beta=dec5_1_beta)
    cat4 = jnp.concatenate([z_enc4, _up2(z_dec5)], axis=-1)

    st = _conv3x3(cat4, bf(dec4_2_w))
    sc, sh = _bn_fold(st, dec4_2_gamma, dec4_2_beta)
    z_dec4 = _conv3x3(st[0], bf(dec4_1_w), sc, sh, mode='fused',
                      gamma=dec4_1_gamma, beta=dec4_1_beta)
    cat3 = jnp.concatenate([z_enc3, _up2(z_dec4)], axis=-1)

    st = _conv3x3(cat3, bf(dec3_2_w))
    sc, sh = _bn_fold(st, dec3_2_gamma, dec3_2_beta)
    st = _conv3x3(st[0], bf(dec3_1_w), sc, sh)
    sc, sh = _bn_fold(st, dec3_1_gamma, dec3_1_beta)
    cat2 = jnp.concatenate([z_enc2, _up2(_apply(st[0], sc, sh))],
                           axis=-1)

    st = _conv3x3(cat2, bf(dec2_2_w))
    sc, sh = _bn_fold(st, dec2_2_gamma, dec2_2_beta)
    st = _conv3x3(st[0], bf(dec2_1_w), sc, sh)
    sc, sh = _bn_fold(st, dec2_1_gamma, dec2_1_beta)
    cat1 = jnp.concatenate([z_enc1, _up2(_apply(st[0], sc, sh))],
                           axis=-1)

    st = _conv3x3(cat1, bf(dec1_2_w))
    sc, sh = _bn_fold(st, dec1_2_gamma, dec1_2_beta)
    out = _conv3x3(st[0], bf(dec1_1_w), sc, sh, mode='final', alpha=0.1)
    return jnp.transpose(out, (0, 3, 1, 2))


# speedup vs baseline: 1.5763x; 1.5763x over previous
import functools

import jax
import jax.numpy as jnp
from jax import lax
from jax.experimental import pallas as pl
from jax.experimental.pallas import tpu as pltpu

_EPS = 1e-5
_BF = jnp.bfloat16
_TH = {256: 4, 128: 8, 64: 16, 32: 32, 16: 16}


def _halo_rows(a, th):
    n, h, w, c = a.shape
    zrow = jnp.zeros((n, 1, w, c), a.dtype)
    top = jnp.concatenate([zrow, a[:, th - 1:h - 1:th]], axis=1)
    bot = jnp.concatenate([a[:, th::th], zrow], axis=1)
    return top, bot


def _conv_body(top_ref, body_ref, bot_ref, scale_ref, shift_ref, w_ref,
               *refs, th, hh, ww, cin, cout, per_tap, affine, mode,
               alpha):
    b = pl.program_id(1)
    slab = jnp.concatenate(
        [top_ref[...], body_ref[...], bot_ref[...]], axis=1)
    if affine:
        t = slab.astype(jnp.float32) * scale_ref[...] + shift_ref[...]
        t = jnp.maximum(t, 0.0)
        g = lax.broadcasted_iota(jnp.int32, (1, th + 2, 1, 1), 1) + b * th - 1
        slab = jnp.where((g >= 0) & (g < hh), t, 0.0).astype(_BF)
    slab = jnp.pad(slab, ((0, 0), (0, 0), (1, 1), (0, 0)))
    m = slab.shape[0] * th * ww
    taps = [(dy, dx) for dy in range(3) for dx in range(3)]
    if per_tap:
        y = jnp.zeros((m, cout), jnp.float32)
        for t_i, (dy, dx) in enumerate(taps):
            p = slab[:, dy:dy + th, dx:dx + ww, :].reshape(m, cin)
            y = y + jnp.dot(p, w_ref[t_i * cin:(t_i + 1) * cin, :],
                            preferred_element_type=jnp.float32)
    else:
        cols = [slab[:, dy:dy + th, dx:dx + ww, :].reshape(m, cin)
                for dy, dx in taps]
        y = jnp.dot(jnp.concatenate(cols, axis=-1), w_ref[...],
                    preferred_element_type=jnp.float32)
    nblk = slab.shape[0]
    if mode == 'final':
        y = jnp.where(y >= 0, y, alpha * y)
        y = 1.0 / (1.0 + jnp.exp(-y))
        refs[0][...] = y.reshape(nblk, th, ww, cout)
    elif mode == 'fused':
        gamma_ref, beta_ref, z_ref = refs
        mean = jnp.sum(y, axis=0, keepdims=True) * (1.0 / m)
        yc = y - mean
        var = jnp.sum(yc * yc, axis=0, keepdims=True) * (1.0 / m)
        sc = gamma_ref[...] * lax.rsqrt(var + _EPS)
        t = yc * sc + beta_ref[...]
        t = jnp.maximum(t, 0.0)
        z_ref[...] = t.reshape(nblk, th, ww, cout).astype(z_ref.dtype)
    else:
        y_ref, s_ref, ss_ref = refs
        y_ref[...] = y.reshape(nblk, th, ww, cout).astype(y_ref.dtype)
        s_ref[...] = jnp.sum(y, axis=0).reshape(1, 1, cout)
        ss_ref[...] = jnp.sum(y * y, axis=0).reshape(1, 1, cout)


def _conv3x3(src, w_bf, scale=None, shift=None, *, mode='stats', alpha=0.0,
             y_dtype=_BF, gamma=None, beta=None):
    n, h, ww, cin = src.shape
    k, cout = w_bf.shape
    fused = mode == 'fused'
    th = h if fused else _TH[h]
    nb = h // th
    per_tap = cin >= 128
    affine = scale is not None
    if not affine:
        scale = jnp.zeros((1, 1, 1, cin), jnp.float32)
        shift = jnp.zeros((1, 1, 1, cin), jnp.float32)
    top, bot = _halo_rows(src, th)

    bn = n if fused else 1
    grid = (1, 1) if fused else (n, nb)
    halo_spec = pl.BlockSpec((bn, 1, ww, cin), lambda i, j: (i, j, 0, 0))
    body_spec = pl.BlockSpec((bn, th, ww, cin), lambda i, j: (i, j, 0, 0))
    vec_spec = pl.BlockSpec((1, 1, 1, cin), lambda i, j: (0, 0, 0, 0))
    w_spec = pl.BlockSpec((k, cout), lambda i, j: (0, 0))
    y_spec = pl.BlockSpec((bn, th, ww, cout), lambda i, j: (i, j, 0, 0))
    st_spec = pl.BlockSpec((1, 1, cout), lambda i, j: (i * nb + j, 0, 0))

    inputs = [top, src, bot, scale, shift, w_bf]
    in_specs = [halo_spec, body_spec, halo_spec, vec_spec, vec_spec, w_spec]
    if mode == 'final':
        out_shape = jax.ShapeDtypeStruct((n, h, ww, cout), jnp.float32)
        out_specs = y_spec
    elif fused:
        g_spec = pl.BlockSpec((1, cout), lambda i, j: (0, 0))
        inputs += [gamma.reshape(1, cout).astype(jnp.float32),
                   beta.reshape(1, cout).astype(jnp.float32)]
        in_specs += [g_spec, g_spec]
        out_shape = jax.ShapeDtypeStruct((n, h, ww, cout), _BF)
        out_specs = y_spec
    else:
        out_shape = (jax.ShapeDtypeStruct((n, h, ww, cout), y_dtype),
                     jax.ShapeDtypeStruct((n * nb, 1, cout), jnp.float32),
                     jax.ShapeDtypeStruct((n * nb, 1, cout), jnp.float32))
        out_specs = (y_spec, st_spec, st_spec)

    body = functools.partial(
        _conv_body, th=th, hh=h, ww=ww, cin=cin, cout=cout,
        per_tap=per_tap, affine=affine, mode=mode, alpha=alpha)
    return pl.pallas_call(
        body,
        grid=grid,
        in_specs=in_specs,
        out_specs=out_specs,
        out_shape=out_shape,
        compiler_params=pltpu.CompilerParams(
            dimension_semantics=("parallel", "parallel"),
            vmem_limit_bytes=100 * 1024 * 1024),
    )(*inputs)


def _bn_fold(stats, gamma, beta):
    y, s, ss = stats
    m = y.shape[0] * y.shape[1] * y.shape[2]
    mean = jnp.sum(s, axis=(0, 1)) / m
    var = jnp.maximum(jnp.sum(ss, axis=(0, 1)) / m - mean * mean, 0.0)
    sc = gamma * lax.rsqrt(var + _EPS)
    sh = beta - mean * sc
    return sc.reshape(1, 1, 1, -1), sh.reshape(1, 1, 1, -1)


def _apply_body(y_ref, sc_ref, sh_ref, z_ref):
    t = y_ref[...].astype(jnp.float32) * sc_ref[...] + sh_ref[...]
    z_ref[...] = jnp.maximum(t, 0.0).astype(z_ref.dtype)


def _apply(y, sc, sh):
    n, h, ww, c = y.shape
    th = _TH[h]
    blk = pl.BlockSpec((1, th, ww, c), lambda i, j: (i, j, 0, 0))
    vec = pl.BlockSpec((1, 1, 1, c), lambda i, j: (0, 0, 0, 0))
    return pl.pallas_call(
        _apply_body,
        grid=(n, h // th),
        in_specs=[blk, vec, vec],
        out_specs=blk,
        out_shape=jax.ShapeDtypeStruct((n, h, ww, c), _BF),
        compiler_params=pltpu.CompilerParams(
            dimension_semantics=("parallel", "parallel")),
    )(y, sc, sh)


def _pool2(z):
    n, h, w, c = z.shape
    return (z.astype(jnp.float32)
            .reshape(n, h // 2, 2, w // 2, 2, c).mean(axis=(2, 4))
            .astype(_BF))


def _up2(z):
    return jnp.repeat(jnp.repeat(z, 2, axis=1), 2, axis=2)


def kernel(x, enc1_1_w, enc1_1_gamma, enc1_1_beta, enc1_2_w, enc1_2_gamma, enc1_2_beta, enc2_1_w, enc2_1_gamma, enc2_1_beta, enc2_2_w, enc2_2_gamma, enc2_2_beta, enc3_1_w, enc3_1_gamma, enc3_1_beta, enc3_2_w, enc3_2_gamma, enc3_2_beta, enc4_1_w, enc4_1_gamma, enc4_1_beta, enc4_2_w, enc4_2_gamma, enc4_2_beta, enc5_1_w, enc5_1_gamma, enc5_1_beta, dec5_1_w, dec5_1_gamma, dec5_1_beta, dec4_2_w, dec4_2_gamma, dec4_2_beta, dec4_1_w, dec4_1_gamma, dec4_1_beta, dec3_2_w, dec3_2_gamma, dec3_2_beta, dec3_1_w, dec3_1_gamma, dec3_1_beta, dec2_2_w, dec2_2_gamma, dec2_2_beta, dec2_1_w, dec2_1_gamma, dec2_1_beta, dec1_2_w, dec1_2_gamma, dec1_2_beta, dec1_1_w):
    bf = lambda a: a.astype(_BF)
    xin = bf(jnp.transpose(x, (0, 2, 3, 1)))

    st = _conv3x3(xin, bf(enc1_1_w))
    sc, sh = _bn_fold(st, enc1_1_gamma, enc1_1_beta)
    st = _conv3x3(st[0], bf(enc1_2_w), sc, sh)
    sc, sh = _bn_fold(st, enc1_2_gamma, enc1_2_beta)
    z_enc1 = _apply(st[0], sc, sh)

    st = _conv3x3(_pool2(z_enc1), bf(enc2_1_w))
    sc, sh = _bn_fold(st, enc2_1_gamma, enc2_1_beta)
    st = _conv3x3(st[0], bf(enc2_2_w), sc, sh)
    sc, sh = _bn_fold(st, enc2_2_gamma, enc2_2_beta)
    z_enc2 = _apply(st[0], sc, sh)

    st = _conv3x3(_pool2(z_enc2), bf(enc3_1_w))
    sc, sh = _bn_fold(st, enc3_1_gamma, enc3_1_beta)
    st = _conv3x3(st[0], bf(enc3_2_w), sc, sh)
    sc, sh = _bn_fold(st, enc3_2_gamma, enc3_2_beta)
    z_enc3 = _apply(st[0], sc, sh)

    st = _conv3x3(_pool2(z_enc3), bf(enc4_1_w))
    sc, sh = _bn_fold(st, enc4_1_gamma, enc4_1_beta)
    st = _conv3x3(st[0], bf(enc4_2_w), sc, sh)
    sc, sh = _bn_fold(st, enc4_2_gamma, enc4_2_beta)
    z_enc4 = _apply(st[0], sc, sh)

    z_enc5 = _conv3x3(_pool2(z_enc4), bf(enc5_1_w), mode='fused',
                      gamma=enc5_1_gamma, beta=enc5_1_beta)

    z_dec5 = _conv3x3(z_enc5, bf(dec5_1_w), mode='fused',
                      gamma=dec5_1_gamma, beta=dec5_1_beta)
    cat4 = jnp.concatenate([z_enc4, _up2(z_dec5)], axis=-1)

    st = _conv3x3(cat4, bf(dec4_2_w))
    sc, sh = _bn_fold(st, dec4_2_gamma, dec4_2_beta)
    z_dec4 = _conv3x3(st[0], bf(dec4_1_w), sc, sh, mode='fused',
                      gamma=dec4_1_gamma, beta=dec4_1_beta)
    cat3 = jnp.concatenate([z_enc3, _up2(z_dec4)], axis=-1)

    st = _conv3x3(cat3, bf(dec3_2_w))
    sc, sh = _bn_fold(st, dec3_2_gamma, dec3_2_beta)
    st = _conv3x3(st[0], bf(dec3_1_w), sc, sh)
    sc, sh = _bn_fold(st, dec3_1_gamma, dec3_1_beta)
    cat2 = jnp.concatenate([z_enc2, _up2(_apply(st[0], sc, sh))],
                           axis=-1)

    st = _conv3x3(cat2, bf(dec2_2_w))
    sc, sh = _bn_fold(st, dec2_2_gamma, dec2_2_beta)
    st = _conv3x3(st[0], bf(dec2_1_w), sc, sh)
    sc, sh = _bn_fold(st, dec2_1_gamma, dec2_1_beta)
    cat1 = jnp.concatenate([z_enc1, _up2(_apply(st[0], sc, sh))],
                           axis=-1)

    st = _conv3x3(cat1, bf(dec1_2_w))
    sc, sh = _bn_fold(st, dec1_2_gamma, dec1_2_beta)
    out = _conv3x3(st[0], bf(dec1_1_w), sc, sh, mode='final', alpha=0.1)
    return jnp.transpose(out, (0, 3, 1, 2))


# speedup vs baseline: 2.2217x; 1.4094x over previous
import functools

import jax
import jax.numpy as jnp
from jax import lax
from jax.experimental import pallas as pl
from jax.experimental.pallas import tpu as pltpu

_EPS = 1e-5
_BF = jnp.bfloat16
_TH = {256: 16, 128: 16, 64: 32, 32: 32, 16: 16}


def _halo_rows(a, th):
    n, h, w, c = a.shape
    zrow = jnp.zeros((n, 1, w, c), a.dtype)
    top = jnp.concatenate([zrow, a[:, th - 1:h - 1:th]], axis=1)
    bot = jnp.concatenate([a[:, th::th], zrow], axis=1)
    return top, bot


def _conv_body(top_ref, body_ref, bot_ref, scale_ref, shift_ref, w_ref,
               *refs, th, hh, ww, cin, cout, per_tap, affine, mode,
               alpha):
    b = pl.program_id(1)
    slab = jnp.concatenate(
        [top_ref[...], body_ref[...], bot_ref[...]], axis=1)
    if affine:
        t = slab.astype(jnp.float32) * scale_ref[...] + shift_ref[...]
        t = jnp.maximum(t, 0.0)
        g = lax.broadcasted_iota(jnp.int32, (1, th + 2, 1, 1), 1) + b * th - 1
        slab = jnp.where((g >= 0) & (g < hh), t, 0.0).astype(_BF)
    slab = jnp.pad(slab, ((0, 0), (0, 0), (1, 1), (0, 0)))
    m = slab.shape[0] * th * ww
    taps = [(dy, dx) for dy in range(3) for dx in range(3)]
    if per_tap:
        y = jnp.zeros((m, cout), jnp.float32)
        for t_i, (dy, dx) in enumerate(taps):
            p = slab[:, dy:dy + th, dx:dx + ww, :].reshape(m, cin)
            y = y + jnp.dot(p, w_ref[t_i * cin:(t_i + 1) * cin, :],
                            preferred_element_type=jnp.float32)
    else:
        cols = [slab[:, dy:dy + th, dx:dx + ww, :].reshape(m, cin)
                for dy, dx in taps]
        y = jnp.dot(jnp.concatenate(cols, axis=-1), w_ref[...],
                    preferred_element_type=jnp.float32)
    nblk = slab.shape[0]
    if mode == 'final':
        y = jnp.where(y >= 0, y, alpha * y)
        y = 1.0 / (1.0 + jnp.exp(-y))
        refs[0][...] = y.reshape(nblk, th, ww, cout)
    elif mode == 'fused':
        gamma_ref, beta_ref, z_ref = refs
        mean = jnp.sum(y, axis=0, keepdims=True) * (1.0 / m)
        yc = y - mean
        var = jnp.sum(yc * yc, axis=0, keepdims=True) * (1.0 / m)
        sc = gamma_ref[...] * lax.rsqrt(var + _EPS)
        t = yc * sc + beta_ref[...]
        t = jnp.maximum(t, 0.0)
        z_ref[...] = t.reshape(nblk, th, ww, cout).astype(z_ref.dtype)
    else:
        y_ref, s_ref, ss_ref = refs
        y_ref[...] = y.reshape(nblk, th, ww, cout).astype(y_ref.dtype)
        s_ref[...] = jnp.sum(y, axis=0).reshape(1, 1, cout)
        ss_ref[...] = jnp.sum(y * y, axis=0).reshape(1, 1, cout)


def _conv3x3(src, w_bf, scale=None, shift=None, *, mode='stats', alpha=0.0,
             y_dtype=_BF, gamma=None, beta=None):
    n, h, ww, cin = src.shape
    k, cout = w_bf.shape
    fused = mode == 'fused'
    th = h if fused else _TH[h]
    nb = h // th
    per_tap = cin >= 128
    affine = scale is not None
    if not affine:
        scale = jnp.zeros((1, 1, 1, cin), jnp.float32)
        shift = jnp.zeros((1, 1, 1, cin), jnp.float32)
    top, bot = _halo_rows(src, th)

    bn = n if fused else 1
    grid = (1, 1) if fused else (n, nb)
    halo_spec = pl.BlockSpec((bn, 1, ww, cin), lambda i, j: (i, j, 0, 0))
    body_spec = pl.BlockSpec((bn, th, ww, cin), lambda i, j: (i, j, 0, 0))
    vec_spec = pl.BlockSpec((1, 1, 1, cin), lambda i, j: (0, 0, 0, 0))
    w_spec = pl.BlockSpec((k, cout), lambda i, j: (0, 0))
    y_spec = pl.BlockSpec((bn, th, ww, cout), lambda i, j: (i, j, 0, 0))
    st_spec = pl.BlockSpec((1, 1, cout), lambda i, j: (i * nb + j, 0, 0))

    inputs = [top, src, bot, scale, shift, w_bf]
    in_specs = [halo_spec, body_spec, halo_spec, vec_spec, vec_spec, w_spec]
    if mode == 'final':
        out_shape = jax.ShapeDtypeStruct((n, h, ww, cout), jnp.float32)
        out_specs = y_spec
    elif fused:
        g_spec = pl.BlockSpec((1, cout), lambda i, j: (0, 0))
        inputs += [gamma.reshape(1, cout).astype(jnp.float32),
                   beta.reshape(1, cout).astype(jnp.float32)]
        in_specs += [g_spec, g_spec]
        out_shape = jax.ShapeDtypeStruct((n, h, ww, cout), _BF)
        out_specs = y_spec
    else:
        out_shape = (jax.ShapeDtypeStruct((n, h, ww, cout), y_dtype),
                     jax.ShapeDtypeStruct((n * nb, 1, cout), jnp.float32),
                     jax.ShapeDtypeStruct((n * nb, 1, cout), jnp.float32))
        out_specs = (y_spec, st_spec, st_spec)

    body = functools.partial(
        _conv_body, th=th, hh=h, ww=ww, cin=cin, cout=cout,
        per_tap=per_tap, affine=affine, mode=mode, alpha=alpha)
    return pl.pallas_call(
        body,
        grid=grid,
        in_specs=in_specs,
        out_specs=out_specs,
        out_shape=out_shape,
        compiler_params=pltpu.CompilerParams(
            dimension_semantics=("parallel", "parallel"),
            vmem_limit_bytes=100 * 1024 * 1024),
    )(*inputs)


def _bn_fold(stats, gamma, beta):
    y, s, ss = stats
    m = y.shape[0] * y.shape[1] * y.shape[2]
    mean = jnp.sum(s, axis=(0, 1)) / m
    var = jnp.maximum(jnp.sum(ss, axis=(0, 1)) / m - mean * mean, 0.0)
    sc = gamma * lax.rsqrt(var + _EPS)
    sh = beta - mean * sc
    return sc.reshape(1, 1, 1, -1), sh.reshape(1, 1, 1, -1)


def _apply_body(y_ref, sc_ref, sh_ref, z_ref):
    t = y_ref[...].astype(jnp.float32) * sc_ref[...] + sh_ref[...]
    z_ref[...] = jnp.maximum(t, 0.0).astype(z_ref.dtype)


def _apply(y, sc, sh):
    n, h, ww, c = y.shape
    th = _TH[h]
    blk = pl.BlockSpec((1, th, ww, c), lambda i, j: (i, j, 0, 0))
    vec = pl.BlockSpec((1, 1, 1, c), lambda i, j: (0, 0, 0, 0))
    return pl.pallas_call(
        _apply_body,
        grid=(n, h // th),
        in_specs=[blk, vec, vec],
        out_specs=blk,
        out_shape=jax.ShapeDtypeStruct((n, h, ww, c), _BF),
        compiler_params=pltpu.CompilerParams(
            dimension_semantics=("parallel", "parallel")),
    )(y, sc, sh)


def _pool2(z):
    n, h, w, c = z.shape
    return (z.astype(jnp.float32)
            .reshape(n, h // 2, 2, w // 2, 2, c).mean(axis=(2, 4))
            .astype(_BF))


def _up2(z):
    return jnp.repeat(jnp.repeat(z, 2, axis=1), 2, axis=2)


def kernel(x, enc1_1_w, enc1_1_gamma, enc1_1_beta, enc1_2_w, enc1_2_gamma, enc1_2_beta, enc2_1_w, enc2_1_gamma, enc2_1_beta, enc2_2_w, enc2_2_gamma, enc2_2_beta, enc3_1_w, enc3_1_gamma, enc3_1_beta, enc3_2_w, enc3_2_gamma, enc3_2_beta, enc4_1_w, enc4_1_gamma, enc4_1_beta, enc4_2_w, enc4_2_gamma, enc4_2_beta, enc5_1_w, enc5_1_gamma, enc5_1_beta, dec5_1_w, dec5_1_gamma, dec5_1_beta, dec4_2_w, dec4_2_gamma, dec4_2_beta, dec4_1_w, dec4_1_gamma, dec4_1_beta, dec3_2_w, dec3_2_gamma, dec3_2_beta, dec3_1_w, dec3_1_gamma, dec3_1_beta, dec2_2_w, dec2_2_gamma, dec2_2_beta, dec2_1_w, dec2_1_gamma, dec2_1_beta, dec1_2_w, dec1_2_gamma, dec1_2_beta, dec1_1_w):
    bf = lambda a: a.astype(_BF)
    xin = bf(jnp.transpose(x, (0, 2, 3, 1)))

    st = _conv3x3(xin, bf(enc1_1_w))
    sc, sh = _bn_fold(st, enc1_1_gamma, enc1_1_beta)
    st = _conv3x3(st[0], bf(enc1_2_w), sc, sh)
    sc, sh = _bn_fold(st, enc1_2_gamma, enc1_2_beta)
    z_enc1 = _apply(st[0], sc, sh)

    st = _conv3x3(_pool2(z_enc1), bf(enc2_1_w))
    sc, sh = _bn_fold(st, enc2_1_gamma, enc2_1_beta)
    st = _conv3x3(st[0], bf(enc2_2_w), sc, sh)
    sc, sh = _bn_fold(st, enc2_2_gamma, enc2_2_beta)
    z_enc2 = _apply(st[0], sc, sh)

    st = _conv3x3(_pool2(z_enc2), bf(enc3_1_w))
    sc, sh = _bn_fold(st, enc3_1_gamma, enc3_1_beta)
    st = _conv3x3(st[0], bf(enc3_2_w), sc, sh)
    sc, sh = _bn_fold(st, enc3_2_gamma, enc3_2_beta)
    z_enc3 = _apply(st[0], sc, sh)

    st = _conv3x3(_pool2(z_enc3), bf(enc4_1_w))
    sc, sh = _bn_fold(st, enc4_1_gamma, enc4_1_beta)
    st = _conv3x3(st[0], bf(enc4_2_w), sc, sh)
    sc, sh = _bn_fold(st, enc4_2_gamma, enc4_2_beta)
    z_enc4 = _apply(st[0], sc, sh)

    z_enc5 = _conv3x3(_pool2(z_enc4), bf(enc5_1_w), mode='fused',
                      gamma=enc5_1_gamma, beta=enc5_1_beta)

    z_dec5 = _conv3x3(z_enc5, bf(dec5_1_w), mode='fused',
                      gamma=dec5_1_gamma, beta=dec5_1_beta)
    cat4 = jnp.concatenate([z_enc4, _up2(z_dec5)], axis=-1)

    st = _conv3x3(cat4, bf(dec4_2_w))
    sc, sh = _bn_fold(st, dec4_2_gamma, dec4_2_beta)
    z_dec4 = _conv3x3(st[0], bf(dec4_1_w), sc, sh, mode='fused',
                      gamma=dec4_1_gamma, beta=dec4_1_beta)
    cat3 = jnp.concatenate([z_enc3, _up2(z_dec4)], axis=-1)

    st = _conv3x3(cat3, bf(dec3_2_w))
    sc, sh = _bn_fold(st, dec3_2_gamma, dec3_2_beta)
    st = _conv3x3(st[0], bf(dec3_1_w), sc, sh)
    sc, sh = _bn_fold(st, dec3_1_gamma, dec3_1_beta)
    cat2 = jnp.concatenate([z_enc2, _up2(_apply(st[0], sc, sh))],
                           axis=-1)

    st = _conv3x3(cat2, bf(dec2_2_w))
    sc, sh = _bn_fold(st, dec2_2_gamma, dec2_2_beta)
    st = _conv3x3(st[0], bf(dec2_1_w), sc, sh)
    sc, sh = _bn_fold(st, dec2_1_gamma, dec2_1_beta)
    cat1 = jnp.concatenate([z_enc1, _up2(_apply(st[0], sc, sh))],
                           axis=-1)

    st = _conv3x3(cat1, bf(dec1_2_w))
    sc, sh = _bn_fold(st, dec1_2_gamma, dec1_2_beta)
    out = _conv3x3(st[0], bf(dec1_1_w), sc, sh, mode='final', alpha=0.1)
    return jnp.transpose(out, (0, 3, 1, 2))


# speedup vs baseline: 2.3905x; 1.0760x over previous
import functools

import jax
import jax.numpy as jnp
from jax import lax
from jax.experimental import pallas as pl
from jax.experimental.pallas import tpu as pltpu

_EPS = 1e-5
_BF = jnp.bfloat16
_TH = {256: 32, 128: 32, 64: 32, 32: 32, 16: 16}


def _halo_rows(a, th):
    n, h, w, c = a.shape
    zrow = jnp.zeros((n, 1, w, c), a.dtype)
    top = jnp.concatenate([zrow, a[:, th - 1:h - 1:th]], axis=1)
    bot = jnp.concatenate([a[:, th::th], zrow], axis=1)
    return top, bot


def _conv_body(top_ref, body_ref, bot_ref, scale_ref, shift_ref, w_ref,
               *refs, th, hh, ww, cin, cout, per_tap, affine, mode,
               alpha):
    b = pl.program_id(1)
    slab = jnp.concatenate(
        [top_ref[...], body_ref[...], bot_ref[...]], axis=1)
    if affine:
        t = slab.astype(jnp.float32) * scale_ref[...] + shift_ref[...]
        t = jnp.maximum(t, 0.0)
        g = lax.broadcasted_iota(jnp.int32, (1, th + 2, 1, 1), 1) + b * th - 1
        slab = jnp.where((g >= 0) & (g < hh), t, 0.0).astype(_BF)
    slab = jnp.pad(slab, ((0, 0), (0, 0), (1, 1), (0, 0)))
    m = slab.shape[0] * th * ww
    taps = [(dy, dx) for dy in range(3) for dx in range(3)]
    if per_tap:
        y = jnp.zeros((m, cout), jnp.float32)
        for t_i, (dy, dx) in enumerate(taps):
            p = slab[:, dy:dy + th, dx:dx + ww, :].reshape(m, cin)
            y = y + jnp.dot(p, w_ref[t_i * cin:(t_i + 1) * cin, :],
                            preferred_element_type=jnp.float32)
    else:
        cols = [slab[:, dy:dy + th, dx:dx + ww, :].reshape(m, cin)
                for dy, dx in taps]
        y = jnp.dot(jnp.concatenate(cols, axis=-1), w_ref[...],
                    preferred_element_type=jnp.float32)
    nblk = slab.shape[0]
    if mode == 'final':
        y = jnp.where(y >= 0, y, alpha * y)
        y = 1.0 / (1.0 + jnp.exp(-y))
        refs[0][...] = y.reshape(nblk, th, ww, cout)
    elif mode == 'fused':
        gamma_ref, beta_ref, z_ref = refs
        mean = jnp.sum(y, axis=0, keepdims=True) * (1.0 / m)
        yc = y - mean
        var = jnp.sum(yc * yc, axis=0, keepdims=True) * (1.0 / m)
        sc = gamma_ref[...] * lax.rsqrt(var + _EPS)
        t = yc * sc + beta_ref[...]
        t = jnp.maximum(t, 0.0)
        z_ref[...] = t.reshape(nblk, th, ww, cout).astype(z_ref.dtype)
    else:
        y_ref, s_ref, ss_ref = refs
        y_ref[...] = y.reshape(nblk, th, ww, cout).astype(y_ref.dtype)
        s_ref[...] = jnp.sum(y, axis=0).reshape(1, 1, cout)
        ss_ref[...] = jnp.sum(y * y, axis=0).reshape(1, 1, cout)


def _conv3x3(src, w_bf, scale=None, shift=None, *, mode='stats', alpha=0.0,
             y_dtype=_BF, gamma=None, beta=None):
    n, h, ww, cin = src.shape
    k, cout = w_bf.shape
    fused = mode == 'fused'
    th = h if fused else _TH[h]
    nb = h // th
    per_tap = cin >= 128
    affine = scale is not None
    if not affine:
        scale = jnp.zeros((1, 1, 1, cin), jnp.float32)
        shift = jnp.zeros((1, 1, 1, cin), jnp.float32)
    top, bot = _halo_rows(src, th)

    bn = n if fused else 1
    grid = (1, 1) if fused else (n, nb)
    halo_spec = pl.BlockSpec((bn, 1, ww, cin), lambda i, j: (i, j, 0, 0))
    body_spec = pl.BlockSpec((bn, th, ww, cin), lambda i, j: (i, j, 0, 0))
    vec_spec = pl.BlockSpec((1, 1, 1, cin), lambda i, j: (0, 0, 0, 0))
    w_spec = pl.BlockSpec((k, cout), lambda i, j: (0, 0))
    y_spec = pl.BlockSpec((bn, th, ww, cout), lambda i, j: (i, j, 0, 0))
    st_spec = pl.BlockSpec((1, 1, cout), lambda i, j: (i * nb + j, 0, 0))

    inputs = [top, src, bot, scale, shift, w_bf]
    in_specs = [halo_spec, body_spec, halo_spec, vec_spec, vec_spec, w_spec]
    if mode == 'final':
        out_shape = jax.ShapeDtypeStruct((n, h, ww, cout), jnp.float32)
        out_specs = y_spec
    elif fused:
        g_spec = pl.BlockSpec((1, cout), lambda i, j: (0, 0))
        inputs += [gamma.reshape(1, cout).astype(jnp.float32),
                   beta.reshape(1, cout).astype(jnp.float32)]
        in_specs += [g_spec, g_spec]
        out_shape = jax.ShapeDtypeStruct((n, h, ww, cout), _BF)
        out_specs = y_spec
    else:
        out_shape = (jax.ShapeDtypeStruct((n, h, ww, cout), y_dtype),
                     jax.ShapeDtypeStruct((n * nb, 1, cout), jnp.float32),
                     jax.ShapeDtypeStruct((n * nb, 1, cout), jnp.float32))
        out_specs = (y_spec, st_spec, st_spec)

    body = functools.partial(
        _conv_body, th=th, hh=h, ww=ww, cin=cin, cout=cout,
        per_tap=per_tap, affine=affine, mode=mode, alpha=alpha)
    return pl.pallas_call(
        body,
        grid=grid,
        in_specs=in_specs,
        out_specs=out_specs,
        out_shape=out_shape,
        compiler_params=pltpu.CompilerParams(
            dimension_semantics=("parallel", "parallel"),
            vmem_limit_bytes=100 * 1024 * 1024),
    )(*inputs)


def _bn_fold(stats, gamma, beta):
    y, s, ss = stats
    m = y.shape[0] * y.shape[1] * y.shape[2]
    mean = jnp.sum(s, axis=(0, 1)) / m
    var = jnp.maximum(jnp.sum(ss, axis=(0, 1)) / m - mean * mean, 0.0)
    sc = gamma * lax.rsqrt(var + _EPS)
    sh = beta - mean * sc
    return sc.reshape(1, 1, 1, -1), sh.reshape(1, 1, 1, -1)


def _apply_body(y_ref, sc_ref, sh_ref, z_ref):
    t = y_ref[...].astype(jnp.float32) * sc_ref[...] + sh_ref[...]
    z_ref[...] = jnp.maximum(t, 0.0).astype(z_ref.dtype)


def _apply(y, sc, sh):
    n, h, ww, c = y.shape
    th = _TH[h]
    blk = pl.BlockSpec((1, th, ww, c), lambda i, j: (i, j, 0, 0))
    vec = pl.BlockSpec((1, 1, 1, c), lambda i, j: (0, 0, 0, 0))
    return pl.pallas_call(
        _apply_body,
        grid=(n, h // th),
        in_specs=[blk, vec, vec],
        out_specs=blk,
        out_shape=jax.ShapeDtypeStruct((n, h, ww, c), _BF),
        compiler_params=pltpu.CompilerParams(
            dimension_semantics=("parallel", "parallel")),
    )(y, sc, sh)


def _pool2(z):
    n, h, w, c = z.shape
    return (z.astype(jnp.float32)
            .reshape(n, h // 2, 2, w // 2, 2, c).mean(axis=(2, 4))
            .astype(_BF))


def _up2(z):
    return jnp.repeat(jnp.repeat(z, 2, axis=1), 2, axis=2)


def kernel(x, enc1_1_w, enc1_1_gamma, enc1_1_beta, enc1_2_w, enc1_2_gamma, enc1_2_beta, enc2_1_w, enc2_1_gamma, enc2_1_beta, enc2_2_w, enc2_2_gamma, enc2_2_beta, enc3_1_w, enc3_1_gamma, enc3_1_beta, enc3_2_w, enc3_2_gamma, enc3_2_beta, enc4_1_w, enc4_1_gamma, enc4_1_beta, enc4_2_w, enc4_2_gamma, enc4_2_beta, enc5_1_w, enc5_1_gamma, enc5_1_beta, dec5_1_w, dec5_1_gamma, dec5_1_beta, dec4_2_w, dec4_2_gamma, dec4_2_beta, dec4_1_w, dec4_1_gamma, dec4_1_beta, dec3_2_w, dec3_2_gamma, dec3_2_beta, dec3_1_w, dec3_1_gamma, dec3_1_beta, dec2_2_w, dec2_2_gamma, dec2_2_beta, dec2_1_w, dec2_1_gamma, dec2_1_beta, dec1_2_w, dec1_2_gamma, dec1_2_beta, dec1_1_w):
    bf = lambda a: a.astype(_BF)
    xin = bf(jnp.transpose(x, (0, 2, 3, 1)))

    st = _conv3x3(xin, bf(enc1_1_w))
    sc, sh = _bn_fold(st, enc1_1_gamma, enc1_1_beta)
    st = _conv3x3(st[0], bf(enc1_2_w), sc, sh)
    sc, sh = _bn_fold(st, enc1_2_gamma, enc1_2_beta)
    z_enc1 = _apply(st[0], sc, sh)

    st = _conv3x3(_pool2(z_enc1), bf(enc2_1_w))
    sc, sh = _bn_fold(st, enc2_1_gamma, enc2_1_beta)
    st = _conv3x3(st[0], bf(enc2_2_w), sc, sh)
    sc, sh = _bn_fold(st, enc2_2_gamma, enc2_2_beta)
    z_enc2 = _apply(st[0], sc, sh)

    st = _conv3x3(_pool2(z_enc2), bf(enc3_1_w))
    sc, sh = _bn_fold(st, enc3_1_gamma, enc3_1_beta)
    st = _conv3x3(st[0], bf(enc3_2_w), sc, sh)
    sc, sh = _bn_fold(st, enc3_2_gamma, enc3_2_beta)
    z_enc3 = _apply(st[0], sc, sh)

    st = _conv3x3(_pool2(z_enc3), bf(enc4_1_w))
    sc, sh = _bn_fold(st, enc4_1_gamma, enc4_1_beta)
    st = _conv3x3(st[0], bf(enc4_2_w), sc, sh)
    sc, sh = _bn_fold(st, enc4_2_gamma, enc4_2_beta)
    z_enc4 = _apply(st[0], sc, sh)

    z_enc5 = _conv3x3(_pool2(z_enc4), bf(enc5_1_w), mode='fused',
                      gamma=enc5_1_gamma, beta=enc5_1_beta)

    z_dec5 = _conv3x3(z_enc5, bf(dec5_1_w), mode='fused',
                      gamma=dec5_1_gamma, beta=dec5_1_beta)
    cat4 = jnp.concatenate([z_enc4, _up2(z_dec5)], axis=-1)

    st = _conv3x3(cat4, bf(dec4_2_w))
    sc, sh = _bn_fold(st, dec4_2_gamma, dec4_2_beta)
    z_dec4 = _conv3x3(st[0], bf(dec4_1_w), sc, sh, mode='fused',
                      gamma=dec4_1_gamma, beta=dec4_1_beta)
    cat3 = jnp.concatenate([z_enc3, _up2(z_dec4)], axis=-1)

    st = _conv3x3(cat3, bf(dec3_2_w))
    sc, sh = _bn_fold(st, dec3_2_gamma, dec3_2_beta)
    st = _conv3x3(st[0], bf(dec3_1_w), sc, sh)
    sc, sh = _bn_fold(st, dec3_1_gamma, dec3_1_beta)
    cat2 = jnp.concatenate([z_enc2, _up2(_apply(st[0], sc, sh))],
                           axis=-1)

    st = _conv3x3(cat2, bf(dec2_2_w))
    sc, sh = _bn_fold(st, dec2_2_gamma, dec2_2_beta)
    st = _conv3x3(st[0], bf(dec2_1_w), sc, sh)
    sc, sh = _bn_fold(st, dec2_1_gamma, dec2_1_beta)
    cat1 = jnp.concatenate([z_enc1, _up2(_apply(st[0], sc, sh))],
                           axis=-1)

    st = _conv3x3(cat1, bf(dec1_2_w))
    sc, sh = _bn_fold(st, dec1_2_gamma, dec1_2_beta)
    out = _conv3x3(st[0], bf(dec1_1_w), sc, sh, mode='final', alpha=0.1)
    return jnp.transpose(out, (0, 3, 1, 2))


# speedup vs baseline: 2.4274x; 1.0154x over previous
import functools

import jax
import jax.numpy as jnp
from jax import lax
from jax.experimental import pallas as pl
from jax.experimental.pallas import tpu as pltpu

_EPS = 1e-5
_BF = jnp.bfloat16
_TH = {256: 64, 128: 32, 64: 32, 32: 32, 16: 16}


def _halo_rows(a, th):
    n, h, w, c = a.shape
    zrow = jnp.zeros((n, 1, w, c), a.dtype)
    top = jnp.concatenate([zrow, a[:, th - 1:h - 1:th]], axis=1)
    bot = jnp.concatenate([a[:, th::th], zrow], axis=1)
    return top, bot


def _conv_body(top_ref, body_ref, bot_ref, scale_ref, shift_ref, w_ref,
               *refs, th, hh, ww, cin, cout, per_tap, affine, mode,
               alpha):
    b = pl.program_id(1)
    slab = jnp.concatenate(
        [top_ref[...], body_ref[...], bot_ref[...]], axis=1)
    if affine:
        t = slab.astype(jnp.float32) * scale_ref[...] + shift_ref[...]
        t = jnp.maximum(t, 0.0)
        g = lax.broadcasted_iota(jnp.int32, (1, th + 2, 1, 1), 1) + b * th - 1
        slab = jnp.where((g >= 0) & (g < hh), t, 0.0).astype(_BF)
    slab = jnp.pad(slab, ((0, 0), (0, 0), (1, 1), (0, 0)))
    m = slab.shape[0] * th * ww
    taps = [(dy, dx) for dy in range(3) for dx in range(3)]
    if per_tap:
        y = jnp.zeros((m, cout), jnp.float32)
        for t_i, (dy, dx) in enumerate(taps):
            p = slab[:, dy:dy + th, dx:dx + ww, :].reshape(m, cin)
            y = y + jnp.dot(p, w_ref[t_i * cin:(t_i + 1) * cin, :],
                            preferred_element_type=jnp.float32)
    else:
        cols = [slab[:, dy:dy + th, dx:dx + ww, :].reshape(m, cin)
                for dy, dx in taps]
        y = jnp.dot(jnp.concatenate(cols, axis=-1), w_ref[...],
                    preferred_element_type=jnp.float32)
    nblk = slab.shape[0]
    if mode == 'final':
        y = jnp.where(y >= 0, y, alpha * y)
        y = 1.0 / (1.0 + jnp.exp(-y))
        refs[0][...] = y.reshape(nblk, th, ww, cout)
    elif mode == 'fused':
        gamma_ref, beta_ref, z_ref = refs
        mean = jnp.sum(y, axis=0, keepdims=True) * (1.0 / m)
        yc = y - mean
        var = jnp.sum(yc * yc, axis=0, keepdims=True) * (1.0 / m)
        sc = gamma_ref[...] * lax.rsqrt(var + _EPS)
        t = yc * sc + beta_ref[...]
        t = jnp.maximum(t, 0.0)
        z_ref[...] = t.reshape(nblk, th, ww, cout).astype(z_ref.dtype)
    else:
        y_ref, s_ref, ss_ref = refs
        y_ref[...] = y.reshape(nblk, th, ww, cout).astype(y_ref.dtype)
        s_ref[...] = jnp.sum(y, axis=0).reshape(1, 1, cout)
        ss_ref[...] = jnp.sum(y * y, axis=0).reshape(1, 1, cout)


def _conv3x3(src, w_bf, scale=None, shift=None, *, mode='stats', alpha=0.0,
             y_dtype=_BF, gamma=None, beta=None):
    n, h, ww, cin = src.shape
    k, cout = w_bf.shape
    fused = mode == 'fused'
    th = h if fused else _TH[h]
    nb = h // th
    per_tap = cin >= 128
    affine = scale is not None
    if not affine:
        scale = jnp.zeros((1, 1, 1, cin), jnp.float32)
        shift = jnp.zeros((1, 1, 1, cin), jnp.float32)
    top, bot = _halo_rows(src, th)

    bn = n if fused else 1
    grid = (1, 1) if fused else (n, nb)
    halo_spec = pl.BlockSpec((bn, 1, ww, cin), lambda i, j: (i, j, 0, 0))
    body_spec = pl.BlockSpec((bn, th, ww, cin), lambda i, j: (i, j, 0, 0))
    vec_spec = pl.BlockSpec((1, 1, 1, cin), lambda i, j: (0, 0, 0, 0))
    w_spec = pl.BlockSpec((k, cout), lambda i, j: (0, 0))
    y_spec = pl.BlockSpec((bn, th, ww, cout), lambda i, j: (i, j, 0, 0))
    st_spec = pl.BlockSpec((1, 1, cout), lambda i, j: (i * nb + j, 0, 0))

    inputs = [top, src, bot, scale, shift, w_bf]
    in_specs = [halo_spec, body_spec, halo_spec, vec_spec, vec_spec, w_spec]
    if mode == 'final':
        out_shape = jax.ShapeDtypeStruct((n, h, ww, cout), jnp.float32)
        out_specs = y_spec
    elif fused:
        g_spec = pl.BlockSpec((1, cout), lambda i, j: (0, 0))
        inputs += [gamma.reshape(1, cout).astype(jnp.float32),
                   beta.reshape(1, cout).astype(jnp.float32)]
        in_specs += [g_spec, g_spec]
        out_shape = jax.ShapeDtypeStruct((n, h, ww, cout), _BF)
        out_specs = y_spec
    else:
        out_shape = (jax.ShapeDtypeStruct((n, h, ww, cout), y_dtype),
                     jax.ShapeDtypeStruct((n * nb, 1, cout), jnp.float32),
                     jax.ShapeDtypeStruct((n * nb, 1, cout), jnp.float32))
        out_specs = (y_spec, st_spec, st_spec)

    body = functools.partial(
        _conv_body, th=th, hh=h, ww=ww, cin=cin, cout=cout,
        per_tap=per_tap, affine=affine, mode=mode, alpha=alpha)
    return pl.pallas_call(
        body,
        grid=grid,
        in_specs=in_specs,
        out_specs=out_specs,
        out_shape=out_shape,
        compiler_params=pltpu.CompilerParams(
            dimension_semantics=("parallel", "parallel"),
            vmem_limit_bytes=100 * 1024 * 1024),
    )(*inputs)


def _bn_fold(stats, gamma, beta):
    y, s, ss = stats
    m = y.shape[0] * y.shape[1] * y.shape[2]
    mean = jnp.sum(s, axis=(0, 1)) / m
    var = jnp.maximum(jnp.sum(ss, axis=(0, 1)) / m - mean * mean, 0.0)
    sc = gamma * lax.rsqrt(var + _EPS)
    sh = beta - mean * sc
    return sc.reshape(1, 1, 1, -1), sh.reshape(1, 1, 1, -1)


def _apply_body(y_ref, sc_ref, sh_ref, z_ref):
    t = y_ref[...].astype(jnp.float32) * sc_ref[...] + sh_ref[...]
    z_ref[...] = jnp.maximum(t, 0.0).astype(z_ref.dtype)


def _apply(y, sc, sh):
    n, h, ww, c = y.shape
    th = _TH[h]
    blk = pl.BlockSpec((1, th, ww, c), lambda i, j: (i, j, 0, 0))
    vec = pl.BlockSpec((1, 1, 1, c), lambda i, j: (0, 0, 0, 0))
    return pl.pallas_call(
        _apply_body,
        grid=(n, h // th),
        in_specs=[blk, vec, vec],
        out_specs=blk,
        out_shape=jax.ShapeDtypeStruct((n, h, ww, c), _BF),
        compiler_params=pltpu.CompilerParams(
            dimension_semantics=("parallel", "parallel")),
    )(y, sc, sh)


def _pool2(z):
    n, h, w, c = z.shape
    return (z.astype(jnp.float32)
            .reshape(n, h // 2, 2, w // 2, 2, c).mean(axis=(2, 4))
            .astype(_BF))


def _up2(z):
    return jnp.repeat(jnp.repeat(z, 2, axis=1), 2, axis=2)


def kernel(x, enc1_1_w, enc1_1_gamma, enc1_1_beta, enc1_2_w, enc1_2_gamma, enc1_2_beta, enc2_1_w, enc2_1_gamma, enc2_1_beta, enc2_2_w, enc2_2_gamma, enc2_2_beta, enc3_1_w, enc3_1_gamma, enc3_1_beta, enc3_2_w, enc3_2_gamma, enc3_2_beta, enc4_1_w, enc4_1_gamma, enc4_1_beta, enc4_2_w, enc4_2_gamma, enc4_2_beta, enc5_1_w, enc5_1_gamma, enc5_1_beta, dec5_1_w, dec5_1_gamma, dec5_1_beta, dec4_2_w, dec4_2_gamma, dec4_2_beta, dec4_1_w, dec4_1_gamma, dec4_1_beta, dec3_2_w, dec3_2_gamma, dec3_2_beta, dec3_1_w, dec3_1_gamma, dec3_1_beta, dec2_2_w, dec2_2_gamma, dec2_2_beta, dec2_1_w, dec2_1_gamma, dec2_1_beta, dec1_2_w, dec1_2_gamma, dec1_2_beta, dec1_1_w):
    bf = lambda a: a.astype(_BF)
    xin = bf(jnp.transpose(x, (0, 2, 3, 1)))

    st = _conv3x3(xin, bf(enc1_1_w))
    sc, sh = _bn_fold(st, enc1_1_gamma, enc1_1_beta)
    st = _conv3x3(st[0], bf(enc1_2_w), sc, sh)
    sc, sh = _bn_fold(st, enc1_2_gamma, enc1_2_beta)
    z_enc1 = _apply(st[0], sc, sh)

    st = _conv3x3(_pool2(z_enc1), bf(enc2_1_w))
    sc, sh = _bn_fold(st, enc2_1_gamma, enc2_1_beta)
    st = _conv3x3(st[0], bf(enc2_2_w), sc, sh)
    sc, sh = _bn_fold(st, enc2_2_gamma, enc2_2_beta)
    z_enc2 = _apply(st[0], sc, sh)

    st = _conv3x3(_pool2(z_enc2), bf(enc3_1_w))
    sc, sh = _bn_fold(st, enc3_1_gamma, enc3_1_beta)
    st = _conv3x3(st[0], bf(enc3_2_w), sc, sh)
    sc, sh = _bn_fold(st, enc3_2_gamma, enc3_2_beta)
    z_enc3 = _apply(st[0], sc, sh)

    st = _conv3x3(_pool2(z_enc3), bf(enc4_1_w))
    sc, sh = _bn_fold(st, enc4_1_gamma, enc4_1_beta)
    st = _conv3x3(st[0], bf(enc4_2_w), sc, sh)
    sc, sh = _bn_fold(st, enc4_2_gamma, enc4_2_beta)
    z_enc4 = _apply(st[0], sc, sh)

    z_enc5 = _conv3x3(_pool2(z_enc4), bf(enc5_1_w), mode='fused',
                      gamma=enc5_1_gamma, beta=enc5_1_beta)

    z_dec5 = _conv3x3(z_enc5, bf(dec5_1_w), mode='fused',
                      gamma=dec5_1_gamma, beta=dec5_1_beta)
    cat4 = jnp.concatenate([z_enc4, _up2(z_dec5)], axis=-1)

    st = _conv3x3(cat4, bf(dec4_2_w))
    sc, sh = _bn_fold(st, dec4_2_gamma, dec4_2_beta)
    z_dec4 = _conv3x3(st[0], bf(dec4_1_w), sc, sh, mode='fused',
                      gamma=dec4_1_gamma, beta=dec4_1_beta)
    cat3 = jnp.concatenate([z_enc3, _up2(z_dec4)], axis=-1)

    st = _conv3x3(cat3, bf(dec3_2_w))
    sc, sh = _bn_fold(st, dec3_2_gamma, dec3_2_beta)
    st = _conv3x3(st[0], bf(dec3_1_w), sc, sh)
    sc, sh = _bn_fold(st, dec3_1_gamma, dec3_1_beta)
    cat2 = jnp.concatenate([z_enc2, _up2(_apply(st[0], sc, sh))],
                           axis=-1)

    st = _conv3x3(cat2, bf(dec2_2_w))
    sc, sh = _bn_fold(st, dec2_2_gamma, dec2_2_beta)
    st = _conv3x3(st[0], bf(dec2_1_w), sc, sh)
    sc, sh = _bn_fold(st, dec2_1_gamma, dec2_1_beta)
    cat1 = jnp.concatenate([z_enc1, _up2(_apply(st[0], sc, sh))],
                           axis=-1)

    st = _conv3x3(cat1, bf(dec1_2_w))
    sc, sh = _bn_fold(st, dec1_2_gamma, dec1_2_beta)
    out = _conv3x3(st[0], bf(dec1_1_w), sc, sh, mode='final', alpha=0.1)
    return jnp.transpose(out, (0, 3, 1, 2))
